# baseline (speedup 1.0000x reference)
import jax
import jax.numpy as jnp
import numpy as np
from jax import lax
from jax.experimental import pallas as pl

_K = 8

_ERFC_P = [
    2.326819970068386e-2, -1.387039388740657e-1, 3.687424674597105e-1,
    -5.824733027278666e-1, 6.210004621745983e-1, -4.944515323274145e-1,
    3.404879937665872e-1, -2.741127028184656e-1, 5.638259427386472e-1,
]
_ERFC_R = [
    -1.047766399936249e+1, 1.297719955372516e+1, -7.495518717768503e+0,
    2.921019019210786e+0, -1.015265279202700e+0, 4.218463358204948e-1,
    -2.820767439740514e-1, 5.641895067754075e-1,
]
_ERF_T = [
    7.853861353153693e-5, -8.010193625184903e-4, 5.188327685732524e-3,
    -2.685381193529856e-2, 1.128358514861418e-1, -3.761262582423300e-1,
    1.128379165726710e+0,
]
_MAXLOG = np.float32(88.72283905206835)


def _poly(x, coeffs):
    p = jnp.zeros_like(x)
    for c in coeffs:
        p = p * x + np.float32(c)
    return p


def _erfc(x):
    abs_x = jnp.abs(x)
    z = jnp.exp(-x * x)
    q = np.float32(1.0) / abs_x
    yq = q * q
    p = jnp.where(abs_x < np.float32(2.0), _poly(yq, _ERFC_P), _poly(yq, _ERFC_R))
    yv = z * q * p
    y_clamp = jnp.where(-x * x < -_MAXLOG, np.float32(0.0), yv)
    erfc_big = jnp.where(x < np.float32(0.0), np.float32(2.0) - y_clamp, y_clamp)
    erf_small = x * _poly(x * x, _ERF_T)
    return jnp.where(abs_x > np.float32(1.0), erfc_big, np.float32(1.0) - erf_small)


def _gelu(x):
    return np.float32(0.5) * x * _erfc(-x * np.float32(np.sqrt(0.5)))


def _conv_body(x_ref, w01_ref, w2_ref, cb_ref, y_ref):
    xb = x_ref[...]
    bt, c, d = xb.shape
    z = jnp.zeros((bt, 1, d), jnp.float32)
    xm1 = jnp.concatenate([z, xb[:, :-1, :]], axis=1)
    xcat01 = jnp.concatenate([xm1, xb], axis=2)
    q = jnp.dot(xcat01.reshape(bt * c, 2 * d), w01_ref[...],
                preferred_element_type=jnp.float32).reshape(bt, c)
    p2 = jnp.dot(xb.reshape(bt * c, d), w2_ref[...],
                 preferred_element_type=jnp.float32).reshape(bt, c)
    z1 = jnp.zeros((bt, 1), jnp.float32)
    p2s = jnp.concatenate([p2[:, 1:], z1], axis=1)
    y_ref[...] = (q + p2s) + cb_ref[0, 0]


def _mlp_body(y_ref, w1_ref, b1_ref, w2_ref, b2_ref, val_ref, idx_ref):
    y = y_ref[...]
    bt = y.shape[0]
    e = w2_ref.shape[1]
    h = jnp.dot(y, w1_ref[...], preferred_element_type=jnp.float32)
    h = h + b1_ref[0, :][None, :]
    g = _gelu(h)
    logits = jnp.dot(g, w2_ref[...], preferred_element_type=jnp.float32)
    logits = logits + b2_ref[0, :][None, :]

    idx64 = lax.broadcasted_iota(jnp.int32, (bt, e), 1)
    vals = logits
    topv, topi = [], []
    for _ in range(_K):
        m = jnp.max(vals, axis=1, keepdims=True)
        cand = jnp.where(vals == m, idx64, e)
        am = jnp.min(cand, axis=1, keepdims=True)
        topv.append(m)
        topi.append(am)
        vals = jnp.where(idx64 == am, -jnp.inf, vals)
    tv = jnp.concatenate(topv, axis=1)
    ti = jnp.concatenate(topi, axis=1)
    ez = jnp.exp(tv - tv[:, 0:1])
    val_ref[...] = ez / jnp.sum(ez, axis=1, keepdims=True)
    idx_ref[...] = ti


def kernel(x, conv_w, conv_b, W1, b1, W2, b2):
    B, C, D = x.shape
    E = W2.shape[1]
    w01 = jnp.concatenate([conv_w[0, :, 0], conv_w[0, :, 1]]).reshape(2 * D, 1)
    w2c = conv_w[0, :, 2].reshape(D, 1)
    cb = conv_b.reshape(1, 1)
    b1r = b1.reshape(1, -1)
    b2r = b2.reshape(1, -1)

    bt_a = 16 if B % 16 == 0 else 8
    y = pl.pallas_call(
        _conv_body,
        grid=(B // bt_a,),
        in_specs=[
            pl.BlockSpec((bt_a, C, D), lambda bi: (bi, 0, 0)),
            pl.BlockSpec((2 * D, 1), lambda bi: (0, 0)),
            pl.BlockSpec((D, 1), lambda bi: (0, 0)),
            pl.BlockSpec((1, 1), lambda bi: (0, 0)),
        ],
        out_specs=pl.BlockSpec((bt_a, C), lambda bi: (bi, 0)),
        out_shape=jax.ShapeDtypeStruct((B, C), jnp.float32),
    )(x, w01, w2c, cb)

    bt_b = min(256, B)
    val, idx = pl.pallas_call(
        _mlp_body,
        grid=(B // bt_b,),
        in_specs=[
            pl.BlockSpec((bt_b, C), lambda bi: (bi, 0)),
            pl.BlockSpec(W1.shape, lambda bi: (0, 0)),
            pl.BlockSpec((1, b1.shape[0]), lambda bi: (0, 0)),
            pl.BlockSpec(W2.shape, lambda bi: (0, 0)),
            pl.BlockSpec((1, E), lambda bi: (0, 0)),
        ],
        out_specs=[
            pl.BlockSpec((bt_b, _K), lambda bi: (bi, 0)),
            pl.BlockSpec((bt_b, _K), lambda bi: (bi, 0)),
        ],
        out_shape=[
            jax.ShapeDtypeStruct((B, _K), jnp.float32),
            jax.ShapeDtypeStruct((B, _K), jnp.int32),
        ],
    )(y, W1, b1r, W2, b2r)
    return (val, idx)
